# baseline (speedup 1.0000x reference)
import functools

import jax
import jax.numpy as jnp
from jax.experimental import pallas as pl
from jax.experimental.pallas import tpu as pltpu
from jax.experimental.pallas import tpu_sc as plsc

N_DIM_EMB = 32
N_ARY_OUT = 64
_PACK = 128 // N_DIM_EMB

_GATHER_WINDOW = 128
_BM = 4096


def _sc_gather128(table128, idxq_flat):
    m = idxq_flat.shape[1]
    mesh = plsc.VectorSubcoreMesh(core_axis_name="core", subcore_axis_name="subcore")

    @functools.partial(
        pl.kernel,
        out_type=jax.ShapeDtypeStruct((m, 128), jnp.float32),
        mesh=mesh,
    )
    def gather_kernel(table_hbm, idx_hbm, out_hbm):
        def body(i_vmem, o_vmem):
            pltpu.sync_copy(table_hbm.at[i_vmem.at[0]], o_vmem)

        pltpu.emit_pipeline(
            body,
            grid=(m // _GATHER_WINDOW,),
            in_specs=[pl.BlockSpec((1, _GATHER_WINDOW), lambda i: (0, i))],
            out_specs=[pl.BlockSpec((_GATHER_WINDOW, 128), lambda i: (i, 0))],
            core_axis_name=("core", "subcore"),
            dimension_semantics=(pltpu.PARALLEL,),
        )(idx_hbm, out_hbm)

    return gather_kernel(table128, idxq_flat)


def _tc_project(ghat, off_col, Ws, b2d):
    m = ghat.shape[0]

    def body(g_ref, off_ref, w_ref, b_ref, o_ref):
        g = g_ref[...]
        off = off_ref[...]
        lane = jax.lax.broadcasted_iota(jnp.int32, g.shape, 1)
        keep = (lane >= off) & (lane < off + N_DIM_EMB)
        g_sel = jnp.where(keep, g, 0.0)
        o_ref[...] = (
            jnp.dot(g_sel, w_ref[...], preferred_element_type=jnp.float32)
            + b_ref[...]
        )

    return pl.pallas_call(
        body,
        grid=(m // _BM,),
        in_specs=[
            pl.BlockSpec((_BM, 128), lambda i: (i, 0)),
            pl.BlockSpec((_BM, 1), lambda i: (i, 0)),
            pl.BlockSpec((128, N_ARY_OUT), lambda i: (0, 0)),
            pl.BlockSpec((1, N_ARY_OUT), lambda i: (0, 0)),
        ],
        out_specs=pl.BlockSpec((_BM, N_ARY_OUT), lambda i: (i, 0)),
        out_shape=jax.ShapeDtypeStruct((m, N_ARY_OUT), jnp.float32),
    )(ghat, off_col, Ws, b2d)


def kernel(indices, table, W, b):
    batch, n_digits = indices.shape
    m = batch * n_digits
    idx_flat = indices.reshape(m)
    idxq = (idx_flat // _PACK).reshape(1, m).astype(jnp.int32)
    off_col = ((idx_flat % _PACK) * N_DIM_EMB).reshape(m, 1).astype(jnp.int32)

    table128 = table.reshape(table.shape[0] // _PACK, 128)
    ghat = _sc_gather128(table128, idxq)

    Ws = jnp.concatenate([W] * _PACK, axis=0)
    logits = _tc_project(ghat, off_col, Ws, b.reshape(1, N_ARY_OUT))
    return logits.reshape(batch, n_digits, N_ARY_OUT)


# speedup vs baseline: 1.9119x; 1.9119x over previous
import functools

import jax
import jax.numpy as jnp
from jax.experimental import pallas as pl
from jax.experimental.pallas import tpu as pltpu
from jax.experimental.pallas import tpu_sc as plsc

N_DIM_EMB = 32
N_ARY_OUT = 64

_GATHER_WINDOW = 128
_BQ = 4096
_BB = 2048


def _tc_project_table(tableT, W, b2d):
    v = tableT.shape[1]

    def body(t_ref, w_ref, b_ref, o_ref):
        res = jax.lax.dot_general(
            t_ref[...], w_ref[...],
            dimension_numbers=(((0,), (0,)), ((), ())),
            preferred_element_type=jnp.float32,
        ) + b_ref[...]
        o_ref[...] = jnp.concatenate(
            [res, jnp.zeros((_BQ, N_ARY_OUT), jnp.float32)], axis=1)

    nblk = (v + _BQ - 1) // _BQ
    return pl.pallas_call(
        body,
        grid=(nblk,),
        in_specs=[
            pl.BlockSpec((N_DIM_EMB, _BQ), lambda i: (0, i)),
            pl.BlockSpec((N_DIM_EMB, N_ARY_OUT), lambda i: (0, 0)),
            pl.BlockSpec((1, N_ARY_OUT), lambda i: (0, 0)),
        ],
        out_specs=pl.BlockSpec((_BQ, 2 * N_ARY_OUT), lambda i: (i, 0)),
        out_shape=jax.ShapeDtypeStruct((v, 2 * N_ARY_OUT), jnp.float32),
    )(tableT, W, b2d)


def _sc_gather(tw, idx_flat):
    m = idx_flat.shape[1]
    mesh = plsc.VectorSubcoreMesh(core_axis_name="core", subcore_axis_name="subcore")

    @functools.partial(
        pl.kernel,
        out_type=jax.ShapeDtypeStruct((m, 2 * N_ARY_OUT), jnp.float32),
        mesh=mesh,
    )
    def gather_kernel(tw_hbm, idx_hbm, out_hbm):
        def body(i_vmem, o_vmem):
            pltpu.sync_copy(tw_hbm.at[i_vmem.at[0]], o_vmem)

        pltpu.emit_pipeline(
            body,
            grid=(m // _GATHER_WINDOW,),
            in_specs=[pl.BlockSpec((1, _GATHER_WINDOW), lambda i: (0, i))],
            out_specs=[pl.BlockSpec((_GATHER_WINDOW, 2 * N_ARY_OUT), lambda i: (i, 0))],
            core_axis_name=("core", "subcore"),
            dimension_semantics=(pltpu.PARALLEL,),
        )(idx_hbm, out_hbm)

    return gather_kernel(tw, idx_flat)


def _tc_transpose(g2, n_digits, batch):
    nj = batch // _BB

    def body(g_ref, o_ref):
        gt = jnp.transpose(g_ref[...], (1, 0))
        o_ref[...] = gt[:N_ARY_OUT, :][None]

    return pl.pallas_call(
        body,
        grid=(n_digits, nj),
        in_specs=[
            pl.BlockSpec((_BB, 2 * N_ARY_OUT), lambda d, j: (d * nj + j, 0)),
        ],
        out_specs=pl.BlockSpec((1, N_ARY_OUT, _BB), lambda d, j: (d, 0, j)),
        out_shape=jax.ShapeDtypeStruct((n_digits, N_ARY_OUT, batch), jnp.float32),
    )(g2)


def kernel(indices, table, W, b):
    batch, n_digits = indices.shape
    m = batch * n_digits

    tableT = table.T
    tw = _tc_project_table(tableT, W, b.reshape(1, N_ARY_OUT))

    idx_flat = indices.T.reshape(1, m)
    g2 = _sc_gather(tw, idx_flat)

    out3 = _tc_transpose(g2, n_digits, batch)
    return out3.transpose(2, 0, 1)


# speedup vs baseline: 2.1866x; 1.1437x over previous
import functools

import jax
import jax.numpy as jnp
from jax.experimental import pallas as pl
from jax.experimental.pallas import tpu as pltpu
from jax.experimental.pallas import tpu_sc as plsc

N_DIM_EMB = 32
N_ARY_OUT = 64

_GATHER_WINDOW = 128
_BQ = 4096
_BB = 2048


def _tc_project_table(tableT, W, b2d):
    v = tableT.shape[1]
    nblk = (v + 2 * _BQ - 1) // (2 * _BQ)

    def body(t_ref, w_ref, b_ref, o_ref):
        res = jax.lax.dot_general(
            t_ref[...], w_ref[...],
            dimension_numbers=(((0,), (0,)), ((), ())),
            preferred_element_type=jnp.float32,
        ) + b_ref[...]
        o_ref[...] = jnp.concatenate([res[:_BQ], res[_BQ:]], axis=1)

    return pl.pallas_call(
        body,
        grid=(nblk,),
        in_specs=[
            pl.BlockSpec((N_DIM_EMB, 2 * _BQ), lambda i: (0, i)),
            pl.BlockSpec((N_DIM_EMB, N_ARY_OUT), lambda i: (0, 0)),
            pl.BlockSpec((1, N_ARY_OUT), lambda i: (0, 0)),
        ],
        out_specs=pl.BlockSpec((_BQ, 2 * N_ARY_OUT), lambda i: (i, 0)),
        out_shape=jax.ShapeDtypeStruct((nblk * _BQ, 2 * N_ARY_OUT), jnp.float32),
    )(tableT, W, b2d)


def _sc_gather(tw2, idx_flat):
    m = idx_flat.shape[1]
    mesh = plsc.VectorSubcoreMesh(core_axis_name="core", subcore_axis_name="subcore")

    @functools.partial(
        pl.kernel,
        out_type=jax.ShapeDtypeStruct((m, 2 * N_ARY_OUT), jnp.float32),
        mesh=mesh,
    )
    def gather_kernel(tw_hbm, idx_hbm, out_hbm):
        def body(i_vmem, o_vmem):
            pltpu.sync_copy(tw_hbm.at[i_vmem.at[0]], o_vmem)

        pltpu.emit_pipeline(
            body,
            grid=(m // _GATHER_WINDOW,),
            in_specs=[pl.BlockSpec((1, _GATHER_WINDOW), lambda i: (0, i))],
            out_specs=[pl.BlockSpec((_GATHER_WINDOW, 2 * N_ARY_OUT), lambda i: (i, 0))],
            core_axis_name=("core", "subcore"),
            dimension_semantics=(pltpu.PARALLEL,),
        )(idx_hbm, out_hbm)

    return gather_kernel(tw2, idx_flat)


def _tc_select_transpose(g2, parityT, n_digits, batch):
    nj = batch // _BB

    def body(g_ref, p_ref, o_ref):
        gt = jnp.transpose(g_ref[...], (1, 0))
        par = p_ref[0]
        sel = jnp.where(par == 0, gt[:N_ARY_OUT, :], gt[N_ARY_OUT:, :])
        o_ref[...] = sel[None]

    return pl.pallas_call(
        body,
        grid=(n_digits, nj),
        in_specs=[
            pl.BlockSpec((_BB, 2 * N_ARY_OUT), lambda d, j: (d * nj + j, 0)),
            pl.BlockSpec((1, 1, _BB), lambda d, j: (d, 0, j)),
        ],
        out_specs=pl.BlockSpec((1, N_ARY_OUT, _BB), lambda d, j: (d, 0, j)),
        out_shape=jax.ShapeDtypeStruct((n_digits, N_ARY_OUT, batch), jnp.float32),
    )(g2, parityT)


def kernel(indices, table, W, b):
    batch, n_digits = indices.shape
    m = batch * n_digits

    tableT = table.T
    tw2 = _tc_project_table(tableT, W, b.reshape(1, N_ARY_OUT))

    idxT = indices.T
    idxq = (((idxT >> 13) << 12) | (idxT & 4095)).reshape(1, m)
    parityT = ((idxT >> 12) & 1).reshape(n_digits, 1, batch)

    g2 = _sc_gather(tw2, idxq)
    out3 = _tc_select_transpose(g2, parityT, n_digits, batch)
    return out3.transpose(2, 0, 1)


# speedup vs baseline: 2.6225x; 1.1994x over previous
import functools

import jax
import jax.numpy as jnp
from jax.experimental import pallas as pl
from jax.experimental.pallas import tpu as pltpu
from jax.experimental.pallas import tpu_sc as plsc

N_DIM_EMB = 32
N_ARY_OUT = 64

_GATHER_WINDOW = 128
_BQ = 8192
_BB = 4096
_NCHUNK = 4


def _tc_project_table(tableT, W, b2d):
    v = tableT.shape[1]
    nblk = (v + 2 * _BQ - 1) // (2 * _BQ)

    def body(t_ref, w_ref, b_ref, o_ref):
        res = jax.lax.dot_general(
            t_ref[...], w_ref[...],
            dimension_numbers=(((0,), (0,)), ((), ())),
            preferred_element_type=jnp.float32,
        ) + b_ref[...]
        o_ref[...] = jnp.concatenate([res[:_BQ], res[_BQ:]], axis=1)

    return pl.pallas_call(
        body,
        grid=(nblk,),
        in_specs=[
            pl.BlockSpec((N_DIM_EMB, 2 * _BQ), lambda i: (0, i)),
            pl.BlockSpec((N_DIM_EMB, N_ARY_OUT), lambda i: (0, 0)),
            pl.BlockSpec((1, N_ARY_OUT), lambda i: (0, 0)),
        ],
        out_specs=pl.BlockSpec((_BQ, 2 * N_ARY_OUT), lambda i: (i, 0)),
        out_shape=jax.ShapeDtypeStruct((nblk * _BQ, 2 * N_ARY_OUT), jnp.float32),
    )(tableT, W, b2d)


def _sc_gather(tw2, idx_chunk):
    m = idx_chunk.shape[1]
    mesh = plsc.VectorSubcoreMesh(core_axis_name="core", subcore_axis_name="subcore")

    @functools.partial(
        pl.kernel,
        out_type=jax.ShapeDtypeStruct((m, 2 * N_ARY_OUT), jnp.float32),
        mesh=mesh,
    )
    def gather_kernel(tw_hbm, idx_hbm, out_hbm):
        def body(i_vmem, o_vmem):
            pltpu.sync_copy(tw_hbm.at[i_vmem.at[0]], o_vmem)

        pltpu.emit_pipeline(
            body,
            grid=(m // _GATHER_WINDOW,),
            in_specs=[pl.BlockSpec((1, _GATHER_WINDOW), lambda i: (0, i))],
            out_specs=[pl.BlockSpec((_GATHER_WINDOW, 2 * N_ARY_OUT), lambda i: (i, 0))],
            core_axis_name=("core", "subcore"),
            dimension_semantics=(pltpu.PARALLEL,),
        )(idx_hbm, out_hbm)

    return gather_kernel(tw2, idx_chunk)


def _tc_select_transpose_chunk(out3_in, g2s, paritys, s, dchunk, n_digits, batch):
    nj = batch // _BB

    def body(g_ref, p_ref, o_ref):
        gt = jnp.transpose(g_ref[...], (1, 0))
        par = p_ref[0]
        sel = jnp.where(par == 0, gt[:N_ARY_OUT, :], gt[N_ARY_OUT:, :])
        o_ref[...] = sel[None]

    def body_aliased(o_in_ref, g_ref, p_ref, o_ref):
        body(g_ref, p_ref, o_ref)

    data_specs = [
        pl.BlockSpec((_BB, 2 * N_ARY_OUT), lambda d, j: (d * nj + j, 0)),
        pl.BlockSpec((1, 1, _BB), lambda d, j: (d, 0, j)),
    ]
    out_spec = pl.BlockSpec(
        (1, N_ARY_OUT, _BB), lambda d, j: (s * dchunk + d, 0, j))
    out_shape = jax.ShapeDtypeStruct((n_digits, N_ARY_OUT, batch), jnp.float32)

    if out3_in is None:
        return pl.pallas_call(
            body,
            grid=(dchunk, nj),
            in_specs=data_specs,
            out_specs=out_spec,
            out_shape=out_shape,
        )(g2s, paritys)
    return pl.pallas_call(
        body_aliased,
        grid=(dchunk, nj),
        in_specs=[pl.BlockSpec(memory_space=pltpu.MemorySpace.HBM)] + data_specs,
        out_specs=out_spec,
        out_shape=out_shape,
        input_output_aliases={0: 0},
    )(out3_in, g2s, paritys)


def kernel(indices, table, W, b):
    batch, n_digits = indices.shape
    m = batch * n_digits
    dchunk = n_digits // _NCHUNK
    mchunk = m // _NCHUNK

    tableT = table.T
    tw2 = _tc_project_table(tableT, W, b.reshape(1, N_ARY_OUT))

    idxT = indices.T
    idxq = ((idxT // (2 * _BQ)) * _BQ + (idxT % _BQ)).reshape(1, m)
    parityT = ((idxT // _BQ) & 1).reshape(n_digits, 1, batch)

    out3 = None
    for s in range(_NCHUNK):
        g2s = _sc_gather(tw2, idxq[:, s * mchunk:(s + 1) * mchunk])
        out3 = _tc_select_transpose_chunk(
            out3, g2s, parityT[s * dchunk:(s + 1) * dchunk],
            s, dchunk, n_digits, batch)
    return out3.transpose(2, 0, 1)


# speedup vs baseline: 2.6239x; 1.0005x over previous
import functools

import jax
import jax.numpy as jnp
from jax.experimental import pallas as pl
from jax.experimental.pallas import tpu as pltpu
from jax.experimental.pallas import tpu_sc as plsc

N_DIM_EMB = 32
N_ARY_OUT = 64

_GATHER_WINDOW = 128
_BQ = 16384
_BB = 4096
_NCHUNK = 5


def _tc_project_table(tableT, W, b2d):
    v = tableT.shape[1]
    nblk = (v + 2 * _BQ - 1) // (2 * _BQ)

    def body(t_ref, w_ref, b_ref, o_ref):
        res = jax.lax.dot_general(
            t_ref[...], w_ref[...],
            dimension_numbers=(((0,), (0,)), ((), ())),
            preferred_element_type=jnp.float32,
        ) + b_ref[...]
        o_ref[...] = jnp.concatenate([res[:_BQ], res[_BQ:]], axis=1)

    return pl.pallas_call(
        body,
        grid=(nblk,),
        in_specs=[
            pl.BlockSpec((N_DIM_EMB, 2 * _BQ), lambda i: (0, i)),
            pl.BlockSpec((N_DIM_EMB, N_ARY_OUT), lambda i: (0, 0)),
            pl.BlockSpec((1, N_ARY_OUT), lambda i: (0, 0)),
        ],
        out_specs=pl.BlockSpec((_BQ, 2 * N_ARY_OUT), lambda i: (i, 0)),
        out_shape=jax.ShapeDtypeStruct((nblk * _BQ, 2 * N_ARY_OUT), jnp.float32),
    )(tableT, W, b2d)


def _sc_gather(tw2, idx_chunk):
    m = idx_chunk.shape[1]
    mesh = plsc.VectorSubcoreMesh(core_axis_name="core", subcore_axis_name="subcore")

    @functools.partial(
        pl.kernel,
        out_type=jax.ShapeDtypeStruct((m, 2 * N_ARY_OUT), jnp.float32),
        mesh=mesh,
    )
    def gather_kernel(tw_hbm, idx_hbm, out_hbm):
        def body(i_vmem, o_vmem):
            pltpu.sync_copy(tw_hbm.at[i_vmem.at[0]], o_vmem)

        pltpu.emit_pipeline(
            body,
            grid=(m // _GATHER_WINDOW,),
            in_specs=[pl.BlockSpec((1, _GATHER_WINDOW), lambda i: (0, i))],
            out_specs=[pl.BlockSpec((_GATHER_WINDOW, 2 * N_ARY_OUT), lambda i: (i, 0))],
            core_axis_name=("core", "subcore"),
            dimension_semantics=(pltpu.PARALLEL,),
        )(idx_hbm, out_hbm)

    return gather_kernel(tw2, idx_chunk)


def _tc_select_transpose_chunk(out3_in, g2s, paritys, s, dchunk, n_digits, batch):
    nj = batch // _BB

    def body(g_ref, p_ref, o_ref):
        gt = jnp.transpose(g_ref[...], (1, 0))
        par = p_ref[0]
        sel = jnp.where(par == 0, gt[:N_ARY_OUT, :], gt[N_ARY_OUT:, :])
        o_ref[...] = sel[None]

    def body_aliased(o_in_ref, g_ref, p_ref, o_ref):
        body(g_ref, p_ref, o_ref)

    data_specs = [
        pl.BlockSpec((_BB, 2 * N_ARY_OUT), lambda d, j: (d * nj + j, 0)),
        pl.BlockSpec((1, 1, _BB), lambda d, j: (d, 0, j)),
    ]
    out_spec = pl.BlockSpec(
        (1, N_ARY_OUT, _BB), lambda d, j: (s * dchunk + d, 0, j))
    out_shape = jax.ShapeDtypeStruct((n_digits, N_ARY_OUT, batch), jnp.float32)

    if out3_in is None:
        return pl.pallas_call(
            body,
            grid=(dchunk, nj),
            in_specs=data_specs,
            out_specs=out_spec,
            out_shape=out_shape,
        )(g2s, paritys)
    return pl.pallas_call(
        body_aliased,
        grid=(dchunk, nj),
        in_specs=[pl.BlockSpec(memory_space=pltpu.MemorySpace.HBM)] + data_specs,
        out_specs=out_spec,
        out_shape=out_shape,
        input_output_aliases={0: 0},
    )(out3_in, g2s, paritys)


def kernel(indices, table, W, b):
    batch, n_digits = indices.shape
    m = batch * n_digits
    dchunk = n_digits // _NCHUNK
    mchunk = m // _NCHUNK

    tableT = table.T
    tw2 = _tc_project_table(tableT, W, b.reshape(1, N_ARY_OUT))

    idxT = indices.T
    idxq = ((idxT // (2 * _BQ)) * _BQ + (idxT % _BQ)).reshape(1, m)
    parityT = ((idxT // _BQ) & 1).reshape(n_digits, 1, batch)

    out3 = None
    for s in range(_NCHUNK):
        g2s = _sc_gather(tw2, idxq[:, s * mchunk:(s + 1) * mchunk])
        out3 = _tc_select_transpose_chunk(
            out3, g2s, parityT[s * dchunk:(s + 1) * dchunk],
            s, dchunk, n_digits, batch)
    return out3.transpose(2, 0, 1)


# speedup vs baseline: 2.6296x; 1.0022x over previous
import functools

import jax
import jax.numpy as jnp
from jax.experimental import pallas as pl
from jax.experimental.pallas import tpu as pltpu
from jax.experimental.pallas import tpu_sc as plsc

N_DIM_EMB = 32
N_ARY_OUT = 64

_GATHER_WINDOW = 128
_BQ = 16384
_BB = 4096
_NCHUNK = 5


def _tc_project_table(tableT, W, b2d):
    v = tableT.shape[1]
    nblk = (v + 2 * _BQ - 1) // (2 * _BQ)

    def body(t_ref, w_ref, b_ref, o_ref):
        res = jax.lax.dot_general(
            t_ref[...], w_ref[...],
            dimension_numbers=(((0,), (0,)), ((), ())),
            preferred_element_type=jnp.float32,
        ) + b_ref[...]
        o_ref[...] = jnp.concatenate([res[:_BQ], res[_BQ:]], axis=1)

    return pl.pallas_call(
        body,
        grid=(nblk,),
        in_specs=[
            pl.BlockSpec((N_DIM_EMB, 2 * _BQ), lambda i: (0, i)),
            pl.BlockSpec((N_DIM_EMB, N_ARY_OUT), lambda i: (0, 0)),
            pl.BlockSpec((1, N_ARY_OUT), lambda i: (0, 0)),
        ],
        out_specs=pl.BlockSpec((_BQ, 2 * N_ARY_OUT), lambda i: (i, 0)),
        out_shape=jax.ShapeDtypeStruct((nblk * _BQ, 2 * N_ARY_OUT), jnp.float32),
    )(tableT, W, b2d)


def _sc_gather(tw2, idx_chunk):
    m = idx_chunk.shape[1]
    mesh = plsc.VectorSubcoreMesh(core_axis_name="core", subcore_axis_name="subcore")

    @functools.partial(
        pl.kernel,
        out_type=jax.ShapeDtypeStruct((m, 2 * N_ARY_OUT), jnp.float32),
        mesh=mesh,
    )
    def gather_kernel(tw_hbm, idx_hbm, out_hbm):
        def body(i_vmem, o_vmem):
            pltpu.sync_copy(tw_hbm.at[i_vmem.at[0]], o_vmem)

        pltpu.emit_pipeline(
            body,
            grid=(m // _GATHER_WINDOW,),
            in_specs=[pl.BlockSpec((1, _GATHER_WINDOW), lambda i: (0, i))],
            out_specs=[pl.BlockSpec((_GATHER_WINDOW, 2 * N_ARY_OUT), lambda i: (i, 0))],
            core_axis_name=("core", "subcore"),
            dimension_semantics=(pltpu.PARALLEL,),
        )(idx_hbm, out_hbm)

    return gather_kernel(tw2, idx_chunk)


def _tc_select_transpose_chunk(out3_in, g2s, paritys, s, dchunk, n_digits, batch):
    nj = batch // _BB

    def body(g_ref, p_ref, o_ref):
        eye = jnp.eye(2 * N_ARY_OUT, dtype=jnp.float32)
        gt = jax.lax.dot_general(
            eye, g_ref[...],
            dimension_numbers=(((1,), (1,)), ((), ())),
            preferred_element_type=jnp.float32,
        )
        par = p_ref[0]
        sel = jnp.where(par == 0, gt[:N_ARY_OUT, :], gt[N_ARY_OUT:, :])
        o_ref[...] = sel[None]

    def body_aliased(o_in_ref, g_ref, p_ref, o_ref):
        body(g_ref, p_ref, o_ref)

    data_specs = [
        pl.BlockSpec((_BB, 2 * N_ARY_OUT), lambda d, j: (d * nj + j, 0)),
        pl.BlockSpec((1, 1, _BB), lambda d, j: (d, 0, j)),
    ]
    out_spec = pl.BlockSpec(
        (1, N_ARY_OUT, _BB), lambda d, j: (s * dchunk + d, 0, j))
    out_shape = jax.ShapeDtypeStruct((n_digits, N_ARY_OUT, batch), jnp.float32)

    if out3_in is None:
        return pl.pallas_call(
            body,
            grid=(dchunk, nj),
            in_specs=data_specs,
            out_specs=out_spec,
            out_shape=out_shape,
        )(g2s, paritys)
    return pl.pallas_call(
        body_aliased,
        grid=(dchunk, nj),
        in_specs=[pl.BlockSpec(memory_space=pltpu.MemorySpace.HBM)] + data_specs,
        out_specs=out_spec,
        out_shape=out_shape,
        input_output_aliases={0: 0},
    )(out3_in, g2s, paritys)


def kernel(indices, table, W, b):
    batch, n_digits = indices.shape
    m = batch * n_digits
    dchunk = n_digits // _NCHUNK
    mchunk = m // _NCHUNK

    tableT = table.T
    tw2 = _tc_project_table(tableT, W, b.reshape(1, N_ARY_OUT))

    idxT = indices.T
    idxq = ((idxT // (2 * _BQ)) * _BQ + (idxT % _BQ)).reshape(1, m)
    parityT = ((idxT // _BQ) & 1).reshape(n_digits, 1, batch)

    out3 = None
    for s in range(_NCHUNK):
        g2s = _sc_gather(tw2, idxq[:, s * mchunk:(s + 1) * mchunk])
        out3 = _tc_select_transpose_chunk(
            out3, g2s, parityT[s * dchunk:(s + 1) * dchunk],
            s, dchunk, n_digits, batch)
    return out3.transpose(2, 0, 1)
